# baseline (speedup 1.0000x reference)
import functools

import jax
import jax.numpy as jnp
from jax import lax
from jax.experimental import pallas as pl
from jax.experimental.pallas import tpu as pltpu
from jax.experimental.pallas import tpu_sc as plsc

NC = 2
NS = 16
NW = NC * NS
LANES = 16

EMB = 64
E_SL = EMB // LANES
C = 64
NPAIR = 5
SEC = 72


def _flatten_seq(seq3d, *, B, L):
    BLK = 128

    def body(in_ref, out_ref):
        y = in_ref[...][:, :, 0]
        z = jnp.zeros((BLK, 128 - L), jnp.int32)
        out_ref[...] = jnp.concatenate([y, z], axis=1)

    return pl.pallas_call(
        body,
        grid=(B // BLK,),
        in_specs=[pl.BlockSpec((BLK, L, 1), lambda i: (i, 0, 0))],
        out_specs=pl.BlockSpec((BLK, 128), lambda i: (i, 0)),
        out_shape=jax.ShapeDtypeStruct((B, 128), jnp.int32),
    )(seq3d)


def _repack_tables(Wt, Ut, Wt_tail, Ut_tail):
    NU = Ut.shape[1]
    NVW = Wt.shape[1]
    nbu = NU // 128
    nbw = NVW // 128
    utail = NU - nbu * 128
    wtail = NVW - nbw * 128

    mesh = plsc.VectorSubcoreMesh(
        core_axis_name="c", subcore_axis_name="s", num_cores=NC, num_subcores=NS
    )

    @functools.partial(
        pl.kernel,
        out_type=(
            jax.ShapeDtypeStruct((NVW // 2, 128), jnp.float32),
            jax.ShapeDtypeStruct((NU // 2, 128), jnp.float32),
        ),
        mesh=mesh,
        compiler_params=pltpu.CompilerParams(needs_layout_passes=False),
        scratch_types=[
            pltpu.VMEM((4, EMB, 128), jnp.float32),
            pltpu.VMEM((2, EMB, 128), jnp.float32),
            pltpu.SemaphoreType.DMA,
            pltpu.SemaphoreType.DMA,
            pltpu.SemaphoreType.DMA,
            pltpu.SemaphoreType.DMA,
            pltpu.SemaphoreType.DMA,
            pltpu.SemaphoreType.DMA,
        ],
    )
    def rk(Wt_hbm, Ut_hbm, Wtt_hbm, Utt_hbm, WL_hbm, UL_hbm,
           inb, outb, si0, si1, si2, si3, so0, so1):
        wid = lax.axis_index("s") * NC + lax.axis_index("c")
        semi = (si0, si1, si2, si3)
        semo = (so0, so1)
        lane = lax.broadcasted_iota(jnp.int32, (LANES,), 0)

        def transpose_rows(p, ob, nrows):
            @pl.loop(0, nrows, unroll=8)
            def _(ro):
                for h in range(2):
                    col = jnp.full((LANES,), 2 * ro + h, jnp.int32)
                    for e in range(E_SL):
                        src = plsc.load_gather(
                            inb.at[p], [e * LANES + lane, col]
                        )
                        outb[ob, ro, pl.ds(h * EMB + e * LANES, LANES)] = src

        def run(src_hbm, dst_hbm, nblk):
            my_n = (nblk - wid + NW - 1) // NW

            def fire(j, p):
                blk = (wid + j * NW) * 128
                pltpu.async_copy(
                    src_hbm.at[:, pl.ds(pl.multiple_of(blk, 128), 128)],
                    inb.at[p],
                    semi[p],
                )

            def drain_in(p):
                pltpu.make_async_copy(
                    src_hbm.at[:, pl.ds(0, 128)], inb.at[p], semi[p]
                ).wait()

            def put(j, ob):
                row = (wid + j * NW) * EMB
                pltpu.async_copy(
                    outb.at[ob],
                    dst_hbm.at[pl.ds(pl.multiple_of(row, 8), EMB)],
                    semo[ob],
                )

            def drain_out(ob):
                pltpu.make_async_copy(
                    outb.at[ob], dst_hbm.at[pl.ds(0, EMB)], semo[ob]
                ).wait()

            for b in range(3):
                pl.when(b < my_n)(functools.partial(fire, b, b))

            @pl.loop(0, (my_n + 3) // 4)
            def _(q):
                for b in range(4):
                    def step(b=b):
                        j = q * 4 + b

                        @pl.when(j + 3 < my_n)
                        def _():
                            fire(j + 3, (b + 3) % 4)

                        drain_in(b)

                        @pl.when(j >= 2)
                        def _():
                            drain_out(b % 2)

                        transpose_rows(b, b % 2, EMB)
                        put(j, b % 2)

                    pl.when(q * 4 + b < my_n)(step)

            pl.when(my_n >= 1)(functools.partial(drain_out, 0))
            pl.when(my_n >= 2)(functools.partial(drain_out, 1))

        run(Wt_hbm, WL_hbm, nbw)
        run(Ut_hbm, UL_hbm, nbu)

        def tail(src_hbm, dst_hbm, nblk, tw):
            pltpu.async_copy(src_hbm, inb.at[0], semi[0]).wait()
            transpose_rows(0, 0, tw // 2)
            pltpu.async_copy(
                outb.at[0, pl.ds(0, tw // 2)],
                dst_hbm.at[pl.ds(pl.multiple_of(nblk * EMB, 8), tw // 2)],
                semo[0],
            ).wait()

        if wtail:
            pl.when(wid == 0)(lambda: tail(Wtt_hbm, WL_hbm, nbw, wtail))
        if utail:
            pl.when(wid == 1)(lambda: tail(Utt_hbm, UL_hbm, nbu, utail))

    return rk(Wt, Ut, Wt_tail, Ut_tail)


def _region_encode(seqp, WL, UL, *, B, L, R):
    TOK = B * L
    b_per_w = B // NW
    per_w = b_per_w * L
    RAD = (R - 1) // 2
    NSL = C + 2 * RAD + 1
    PAD_SLOT = NSL - 1
    NG = (NSL + LANES - 1) // LANES
    n_chunks = per_w // C

    mesh = plsc.VectorSubcoreMesh(
        core_axis_name="c", subcore_axis_name="s", num_cores=NC, num_subcores=NS
    )

    @functools.partial(
        pl.kernel,
        out_type=jax.ShapeDtypeStruct((TOK, EMB), jnp.float32),
        mesh=mesh,
        compiler_params=pltpu.CompilerParams(needs_layout_passes=False),
        scratch_types=[
            pltpu.VMEM((b_per_w, 128), jnp.int32),
            pltpu.VMEM((2, C), jnp.int32),
            pltpu.VMEM((2, SEC), jnp.int32),
            pltpu.VMEM((2, NPAIR * SEC, 128), jnp.float32),
            pltpu.VMEM((2, C, 128), jnp.float32),
            pltpu.VMEM((C, EMB), jnp.float32),
        ]
        + [pltpu.VMEM((SEC,), jnp.int32) for _ in range(2 * NPAIR)]
        + [pltpu.VMEM((C,), jnp.int32) for _ in range(2)]
        + [
            pltpu.SemaphoreType.DMA,
            pltpu.SemaphoreType.DMA,
        ],
    )
    def k(seq_hbm, WL_hbm, UL_hbm, out_hbm,
          seq_v, toks, spar, u_buf, w_buf, out_v, *rest):
        uidx = [rest[p * NPAIR + j] for p in range(2) for j in range(NPAIR)]
        u_idx = lambda p, j: uidx[p * NPAIR + j]
        w_idxs = rest[2 * NPAIR:2 * NPAIR + 2]
        semu0, semu1 = rest[2 * NPAIR + 2:]
        wid = lax.axis_index("s") * NC + lax.axis_index("c")
        base = wid * per_w
        pltpu.sync_copy(seq_hbm.at[pl.ds(wid * b_per_w, b_per_w)], seq_v)

        semu = (semu0, semu1)
        lane = lax.broadcasted_iota(jnp.int32, (LANES,), 0)

        group_starts = list(range(0, NSL - LANES, LANES)) + [SEC - LANES]

        def build_and_fire(c, p):
            for sl0 in group_starts:
                sl = sl0 + lane
                q = jnp.clip(c * C - RAD + sl, 0, per_w - 1)
                v = plsc.load_gather(seq_v, [q // L, lax.rem(q, L)])
                if sl0 + LANES > PAD_SLOT:
                    v = jnp.where(sl == PAD_SLOT, 0, v)
                vb = (v * R) >> 1
                for j in range(NPAIR):
                    u_idx(p, j)[pl.ds(sl0, LANES)] = vb + j
                spar[p, pl.ds(sl0, LANES)] = jnp.bitwise_and(v, 1)
                if sl0 + LANES <= C:
                    cq = c * C + sl
                    tv = plsc.load_gather(seq_v, [cq // L, lax.rem(cq, L)])
                    toks[p, pl.ds(sl0, LANES)] = tv
                    w_idxs[p][pl.ds(sl0, LANES)] = tv >> 1

            for j in range(NPAIR):
                pltpu.async_copy(
                    UL_hbm.at[u_idx(p, j)],
                    u_buf.at[p, pl.ds(j * SEC, SEC)],
                    semu[p],
                )
            pltpu.async_copy(WL_hbm.at[w_idxs[p]], w_buf.at[p], semu[p])

        def drain(p):
            for j in range(NPAIR):
                pltpu.make_async_copy(
                    UL_hbm.at[pl.ds(0, SEC)],
                    u_buf.at[p, pl.ds(j * SEC, SEC)],
                    semu[p],
                ).wait()
            pltpu.make_async_copy(
                UL_hbm.at[pl.ds(0, C)], w_buf.at[p], semu[p]
            ).wait()

        def compute(c, p):
            @pl.loop(0, C // LANES)
            def grp(g):
                g16 = g * LANES
                tv = toks[p, pl.ds(g16, LANES)]
                mvec = jnp.where(tv != 0, 1.0, 0.0).astype(jnp.float32)
                pvs = [
                    plsc.load_gather(spar.at[p], [g16 + RAD + d + lane])
                    for d in range(-RAD, RAD + 1)
                ]
                for cl in range(LANES):
                    l = g16 + cl
                    lpos = lax.rem(c * C + l, L)
                    maskf = mvec[cl]
                    wpar = jnp.bitwise_and(tv[cl], 1)
                    rows = []
                    halfs = []
                    for i in range(R):
                        d = i - RAD
                        par = pvs[d + RAD][cl]
                        lq = lpos + d
                        valid = jnp.logical_and(lq >= 0, lq <= L - 1)
                        slot_s = jnp.where(valid, l + RAD + d, PAD_SLOT)
                        par_s = jnp.where(valid, par, 0)
                        rows.append(((i + par_s) >> 1) * SEC + slot_s)
                        halfs.append(jnp.bitwise_and(i + par_s, 1) * EMB)
                    for e in range(E_SL):
                        w_e = w_buf[p, l, pl.ds(wpar * EMB + e * LANES, LANES)]
                        acc = None
                        for i in range(R):
                            term = (
                                u_buf[
                                    p, rows[i],
                                    pl.ds(halfs[i] + e * LANES, LANES),
                                ]
                                * w_e
                            )
                            acc = (
                                term if acc is None else jnp.maximum(acc, term)
                            )
                        out_v[l, pl.ds(e * LANES, LANES)] = acc * maskf

            pltpu.sync_copy(out_v, out_hbm.at[pl.ds(base + c * C, C)])

        build_and_fire(0, 0)

        @pl.loop(0, (n_chunks - 1) // 2)
        def pair_loop(t):
            c0 = 2 * t
            build_and_fire(c0 + 1, 1)
            drain(0)
            compute(c0, 0)
            build_and_fire(c0 + 2, 0)
            drain(1)
            compute(c0 + 1, 1)

        drain(0)
        compute(n_chunks - 1, 0)

    return k(seqp, WL, UL)


def kernel(seq, W, U):
    B, L, _ = seq.shape
    R = U.shape[0] // W.shape[0]
    nbw = W.shape[0] // 128
    nbu = U.shape[0] // 128
    wt_tail = jnp.pad(
        W[nbw * 128:].T, ((0, 0), (0, 128 - (W.shape[0] - nbw * 128)))
    )
    ut_tail = jnp.pad(
        U[nbu * 128:].T, ((0, 0), (0, 128 - (U.shape[0] - nbu * 128)))
    )
    seqp = _flatten_seq(seq, B=B, L=L)
    WL, UL = _repack_tables(W.T, U.T, wt_tail, ut_tail)
    out = _region_encode(seqp, WL, UL, B=B, L=L, R=R)
    return out.reshape(B, L, 1, EMB)


# speedup vs baseline: 2.1599x; 2.1599x over previous
import functools

import jax
import jax.numpy as jnp
from jax import lax
from jax.experimental import pallas as pl
from jax.experimental.pallas import tpu as pltpu
from jax.experimental.pallas import tpu_sc as plsc

NC = 2
NS = 16
NW = NC * NS
LANES = 16

EMB = 64
E_SL = EMB // LANES
C = 64
NPAIR = 5
SEC = 72


def _flatten_seq(seq3d, *, B, L):
    BLK = 128

    def body(in_ref, out_ref):
        y = in_ref[...][:, :, 0]
        z = jnp.zeros((BLK, 128 - L), jnp.int32)
        out_ref[...] = jnp.concatenate([y, z], axis=1)

    return pl.pallas_call(
        body,
        grid=(B // BLK,),
        in_specs=[pl.BlockSpec((BLK, L, 1), lambda i: (i, 0, 0))],
        out_specs=pl.BlockSpec((BLK, 128), lambda i: (i, 0)),
        out_shape=jax.ShapeDtypeStruct((B, 128), jnp.int32),
    )(seq3d)


def _region_encode(seqp, WL, UL, *, B, L, R):
    TOK = B * L
    b_per_w = B // NW
    per_w = b_per_w * L
    RAD = (R - 1) // 2
    NSL = C + 2 * RAD + 1
    PAD_SLOT = NSL - 1
    NG = (NSL + LANES - 1) // LANES
    n_chunks = per_w // C

    mesh = plsc.VectorSubcoreMesh(
        core_axis_name="c", subcore_axis_name="s", num_cores=NC, num_subcores=NS
    )

    @functools.partial(
        pl.kernel,
        out_type=jax.ShapeDtypeStruct((TOK, EMB), jnp.float32),
        mesh=mesh,
        compiler_params=pltpu.CompilerParams(needs_layout_passes=False),
        scratch_types=[
            pltpu.VMEM((b_per_w, 128), jnp.int32),
            pltpu.VMEM((2, C), jnp.int32),
            pltpu.VMEM((2, SEC), jnp.int32),
            pltpu.VMEM((2, NPAIR * SEC, 128), jnp.float32),
            pltpu.VMEM((2, C, 128), jnp.float32),
            pltpu.VMEM((C, EMB), jnp.float32),
        ]
        + [pltpu.VMEM((SEC,), jnp.int32) for _ in range(2 * NPAIR)]
        + [pltpu.VMEM((C,), jnp.int32) for _ in range(2)]
        + [
            pltpu.SemaphoreType.DMA,
            pltpu.SemaphoreType.DMA,
        ],
    )
    def k(seq_hbm, WL_hbm, UL_hbm, out_hbm,
          seq_v, toks, spar, u_buf, w_buf, out_v, *rest):
        uidx = [rest[p * NPAIR + j] for p in range(2) for j in range(NPAIR)]
        u_idx = lambda p, j: uidx[p * NPAIR + j]
        w_idxs = rest[2 * NPAIR:2 * NPAIR + 2]
        semu0, semu1 = rest[2 * NPAIR + 2:]
        wid = lax.axis_index("s") * NC + lax.axis_index("c")
        base = wid * per_w
        pltpu.sync_copy(seq_hbm.at[pl.ds(wid * b_per_w, b_per_w)], seq_v)

        semu = (semu0, semu1)
        lane = lax.broadcasted_iota(jnp.int32, (LANES,), 0)

        group_starts = list(range(0, NSL - LANES, LANES)) + [SEC - LANES]

        def build_and_fire(c, p):
            for sl0 in group_starts:
                sl = sl0 + lane
                q = jnp.clip(c * C - RAD + sl, 0, per_w - 1)
                v = plsc.load_gather(seq_v, [q // L, lax.rem(q, L)])
                if sl0 + LANES > PAD_SLOT:
                    v = jnp.where(sl == PAD_SLOT, 0, v)
                vb = (v * R) >> 1
                for j in range(NPAIR):
                    u_idx(p, j)[pl.ds(sl0, LANES)] = vb + j
                spar[p, pl.ds(sl0, LANES)] = jnp.bitwise_and(v, 1)
                if sl0 + LANES <= C:
                    cq = c * C + sl
                    tv = plsc.load_gather(seq_v, [cq // L, lax.rem(cq, L)])
                    toks[p, pl.ds(sl0, LANES)] = tv
                    w_idxs[p][pl.ds(sl0, LANES)] = tv >> 1

            for j in range(NPAIR):
                pltpu.async_copy(
                    UL_hbm.at[u_idx(p, j)],
                    u_buf.at[p, pl.ds(j * SEC, SEC)],
                    semu[p],
                )
            pltpu.async_copy(WL_hbm.at[w_idxs[p]], w_buf.at[p], semu[p])

        def drain(p):
            for j in range(NPAIR):
                pltpu.make_async_copy(
                    UL_hbm.at[pl.ds(0, SEC)],
                    u_buf.at[p, pl.ds(j * SEC, SEC)],
                    semu[p],
                ).wait()
            pltpu.make_async_copy(
                UL_hbm.at[pl.ds(0, C)], w_buf.at[p], semu[p]
            ).wait()

        def compute(c, p):
            @pl.loop(0, C // LANES)
            def grp(g):
                g16 = g * LANES
                tv = toks[p, pl.ds(g16, LANES)]
                mvec = jnp.where(tv != 0, 1.0, 0.0).astype(jnp.float32)
                pvs = [
                    plsc.load_gather(spar.at[p], [g16 + RAD + d + lane])
                    for d in range(-RAD, RAD + 1)
                ]
                for cl in range(LANES):
                    l = g16 + cl
                    lpos = lax.rem(c * C + l, L)
                    maskf = mvec[cl]
                    wpar = jnp.bitwise_and(tv[cl], 1)
                    rows = []
                    halfs = []
                    for i in range(R):
                        d = i - RAD
                        par = pvs[d + RAD][cl]
                        lq = lpos + d
                        valid = jnp.logical_and(lq >= 0, lq <= L - 1)
                        slot_s = jnp.where(valid, l + RAD + d, PAD_SLOT)
                        par_s = jnp.where(valid, par, 0)
                        rows.append(((i + par_s) >> 1) * SEC + slot_s)
                        halfs.append(jnp.bitwise_and(i + par_s, 1) * EMB)
                    for e in range(E_SL):
                        w_e = w_buf[p, l, pl.ds(wpar * EMB + e * LANES, LANES)]
                        acc = None
                        for i in range(R):
                            term = (
                                u_buf[
                                    p, rows[i],
                                    pl.ds(halfs[i] + e * LANES, LANES),
                                ]
                                * w_e
                            )
                            acc = (
                                term if acc is None else jnp.maximum(acc, term)
                            )
                        out_v[l, pl.ds(e * LANES, LANES)] = acc * maskf

            pltpu.sync_copy(out_v, out_hbm.at[pl.ds(base + c * C, C)])

        build_and_fire(0, 0)

        @pl.loop(0, (n_chunks - 1) // 2)
        def pair_loop(t):
            c0 = 2 * t
            build_and_fire(c0 + 1, 1)
            drain(0)
            compute(c0, 0)
            build_and_fire(c0 + 2, 0)
            drain(1)
            compute(c0 + 1, 1)

        drain(0)
        compute(n_chunks - 1, 0)

    return k(seqp, WL, UL)


def kernel(seq, W, U):
    B, L, _ = seq.shape
    R = U.shape[0] // W.shape[0]
    seqp = _flatten_seq(seq, B=B, L=L)
    WL = W.reshape(W.shape[0] // 2, 2 * EMB)
    UL = U.reshape(U.shape[0] // 2, 2 * EMB)
    out = _region_encode(seqp, WL, UL, B=B, L=L, R=R)
    return out.reshape(B, L, 1, EMB)
